# baseline (speedup 1.0000x reference)
import jax
import jax.numpy as jnp
from jax.experimental import pallas as pl
from jax.experimental.pallas import tpu as pltpu

N = 10000
D = 64
S = 2
BM = 200
K = N
NS = N // S


def _matmul_block(*refs):
    a_refs = refs[:S]
    e_ref = refs[S]
    o_ref = refs[S + 1]
    for s in range(S):
        o_ref[s] = jnp.dot(
            a_refs[s][0],
            e_ref[...],
            precision=jax.lax.Precision.DEFAULT,
            preferred_element_type=jnp.float32,
        )


def kernel(A_hat, E):
    A3 = A_hat.reshape(S, NS, K)
    in_specs = [
        pl.BlockSpec((1, BM, K), lambda i, s=s: (s, i, 0)) for s in range(S)
    ]
    in_specs.append(pl.BlockSpec((K, D), lambda i: (0, 0)))
    out = pl.pallas_call(
        _matmul_block,
        grid=(NS // BM,),
        in_specs=in_specs,
        out_specs=pl.BlockSpec((S, BM, D), lambda i: (0, i, 0)),
        out_shape=jax.ShapeDtypeStruct((S, NS, D), jnp.float32),
        compiler_params=pltpu.CompilerParams(
            dimension_semantics=("arbitrary",),
        ),
    )(*([A3] * S), E)
    return out.reshape(N, D)
